# Initial kernel scaffold; baseline (speedup 1.0000x reference)
import jax
import jax.numpy as jnp
from jax.experimental import pallas as pl


def kernel(x_word, table):
    raise NotImplementedError("write your pallas kernel here")


# speedup vs baseline: 3.3270x; 3.3270x over previous
import functools

import jax
import jax.numpy as jnp
from jax import lax
from jax.experimental import pallas as pl
from jax.experimental.pallas import tpu as pltpu
from jax.experimental.pallas import tpu_sc as plsc


def _make_gather(N, V, D, NC, NS):
    NW = NC * NS
    C = 128
    n_per_w = N // NW
    nchunks = n_per_w // C
    mesh = plsc.VectorSubcoreMesh(core_axis_name="c", subcore_axis_name="s")

    @functools.partial(
        pl.kernel,
        mesh=mesh,
        out_type=jax.ShapeDtypeStruct((N, D), jnp.float32),
        scratch_types=[
            pltpu.VMEM((nchunks, C), jnp.int32),
            pltpu.VMEM((C, D), jnp.float32),
            pltpu.VMEM((C, D), jnp.float32),
            pltpu.SemaphoreType.DMA,
            pltpu.SemaphoreType.DMA,
        ],
    )
    def gather_kernel(idx_hbm, table_hbm, out_hbm, idx_v, rows0, rows1,
                      sem0, sem1):
        wid = lax.axis_index("s") * NC + lax.axis_index("c")
        row_base = wid * n_per_w
        pltpu.sync_copy(idx_hbm.at[wid], idx_v)
        rows = (rows0, rows1)
        sems = (sem0, sem1)

        def start(c, b):
            pltpu.async_copy(table_hbm.at[idx_v.at[c]], rows[b], sems[b])

        def finish(c, b):
            pltpu.make_async_copy(
                table_hbm.at[idx_v.at[c]], rows[b], sems[b]).wait()
            pltpu.sync_copy(rows[b], out_hbm.at[pl.ds(row_base + c * C, C)])

        start(0, 0)
        start(1, 1)

        @pl.loop(0, nchunks - 2, step=2)
        def _(g):
            for b in range(2):
                finish(g + b, b)
                start(g + 2 + b, b)

        for b in range(2):
            finish(nchunks - 2 + b, b)

    return gather_kernel, NW, nchunks, C


def kernel(x_word, table):
    B, S = x_word.shape
    V, D = table.shape
    N = B * S
    info = plsc.get_sparse_core_info()
    fn, NW, nchunks, C = _make_gather(N, V, D, info.num_cores,
                                      info.num_subcores)
    idx = x_word.reshape(NW, nchunks, C).astype(jnp.int32)
    out = fn(idx, table)
    return out.reshape(B, S, D)


# speedup vs baseline: 3.3568x; 1.0089x over previous
import functools

import jax
import jax.numpy as jnp
from jax import lax
from jax.experimental import pallas as pl
from jax.experimental.pallas import tpu as pltpu
from jax.experimental.pallas import tpu_sc as plsc


def _make_gather(N, V, D, NC, NS):
    NW = NC * NS
    C = 128
    NBUF = 5
    DW = 2
    n_per_w = N // NW
    nchunks = n_per_w // C
    assert nchunks % NBUF == 0 and nchunks > NBUF
    mesh = plsc.VectorSubcoreMesh(core_axis_name="c", subcore_axis_name="s")

    @functools.partial(
        pl.kernel,
        mesh=mesh,
        out_type=jax.ShapeDtypeStruct((N, D), jnp.float32),
        scratch_types=[
            pltpu.VMEM((nchunks, C), jnp.int32),
            [pltpu.VMEM((C, D), jnp.float32)] * NBUF,
            [pltpu.SemaphoreType.DMA] * NBUF,
            [pltpu.SemaphoreType.DMA] * NBUF,
        ],
    )
    def gather_kernel(idx_hbm, table_hbm, out_hbm, idx_v, rows, gsems, wsems):
        wid = lax.axis_index("s") * NC + lax.axis_index("c")
        row_base = wid * n_per_w
        pltpu.sync_copy(idx_hbm.at[wid], idx_v)

        def gather_start(c, b):
            pltpu.async_copy(table_hbm.at[idx_v.at[c]], rows[b], gsems[b])

        def gather_wait(c, b):
            pltpu.make_async_copy(
                table_hbm.at[idx_v.at[c]], rows[b], gsems[b]).wait()

        def write_start(c, b):
            pltpu.async_copy(
                rows[b], out_hbm.at[pl.ds(row_base + c * C, C)], wsems[b])

        def write_wait(c, b):
            pltpu.make_async_copy(
                rows[b], out_hbm.at[pl.ds(row_base + c * C, C)],
                wsems[b]).wait()

        for j in range(NBUF - DW):
            gather_start(j, j)

        @pl.loop(0, nchunks, step=NBUF)
        def _(g):
            for j in range(NBUF):
                c = g + j
                gather_wait(c, j)
                write_start(c, j)
                b2 = (j - DW) % NBUF

                @pl.when(c >= DW)
                def _():
                    write_wait(c - DW, b2)

                @pl.when(c + (NBUF - DW) < nchunks)
                def _():
                    gather_start(c + (NBUF - DW), b2)

        for t in range(DW):
            c = nchunks - DW + t
            write_wait(c, c % NBUF)

    return gather_kernel, NW, nchunks, C


def kernel(x_word, table):
    B, S = x_word.shape
    V, D = table.shape
    N = B * S
    info = plsc.get_sparse_core_info()
    fn, NW, nchunks, C = _make_gather(N, V, D, info.num_cores,
                                      info.num_subcores)
    idx = x_word.reshape(NW, nchunks, C).astype(jnp.int32)
    out = fn(idx, table)
    return out.reshape(B, S, D)


# speedup vs baseline: 10.3738x; 3.0904x over previous
import functools

import jax
import jax.numpy as jnp
from jax import lax
from jax.experimental import pallas as pl
from jax.experimental.pallas import tpu as pltpu
from jax.experimental.pallas import tpu_sc as plsc


def _make_gather(N, V, D, NC, NS):
    NW = NC * NS
    C = 128
    K = 2
    NBUF = 3
    DW = 1
    n_per_w = N // NW
    nchunks = n_per_w // C
    nfills = nchunks // K
    assert n_per_w % (K * C) == 0 and nfills > NBUF
    nfills_main = nfills - (nfills % NBUF)
    PCH = (nchunks + 7) // 8 * 8
    mesh = plsc.VectorSubcoreMesh(core_axis_name="c", subcore_axis_name="s")

    @functools.partial(
        pl.kernel,
        mesh=mesh,
        out_type=jax.ShapeDtypeStruct((N, D), jnp.float32),
        scratch_types=[
            pltpu.VMEM((PCH, C), jnp.int32),
            [pltpu.VMEM((K * C, D), jnp.float32)] * NBUF,
            [pltpu.SemaphoreType.DMA] * NBUF,
            [pltpu.SemaphoreType.DMA] * NBUF,
        ],
    )
    def gather_kernel(idx_hbm, table_hbm, out_hbm, idx_v, rows, gsems, wsems):
        wid = lax.axis_index("s") * NC + lax.axis_index("c")
        row_base = wid * n_per_w
        pltpu.sync_copy(idx_hbm.at[wid], idx_v)

        def gather_start(f, b):
            for t in range(K):
                pltpu.async_copy(table_hbm.at[idx_v.at[K * f + t]],
                                 rows[b].at[pl.ds(t * C, C)], gsems[b])

        def gather_wait(f, b):
            for t in range(K):
                pltpu.make_async_copy(table_hbm.at[idx_v.at[K * f + t]],
                                      rows[b].at[pl.ds(t * C, C)],
                                      gsems[b]).wait()

        def write_start(f, b):
            pltpu.async_copy(
                rows[b], out_hbm.at[pl.ds(row_base + f * K * C, K * C)],
                wsems[b])

        def write_wait(f, b):
            pltpu.make_async_copy(
                rows[b], out_hbm.at[pl.ds(row_base + f * K * C, K * C)],
                wsems[b]).wait()

        for j in range(NBUF - DW):
            gather_start(j, j)

        @pl.loop(0, nfills_main, step=NBUF)
        def _(g):
            for j in range(NBUF):
                f = g + j
                gather_wait(f, j)
                write_start(f, j)
                b2 = (j - DW) % NBUF

                @pl.when(f >= DW)
                def _():
                    write_wait(f - DW, b2)

                @pl.when(f + (NBUF - DW) < nfills)
                def _():
                    gather_start(f + (NBUF - DW), b2)

        for f in range(nfills_main, nfills):
            gather_wait(f, f % NBUF)
            write_start(f, f % NBUF)
            write_wait(f - DW, (f - DW) % NBUF)
        for t in range(DW):
            f = nfills - DW + t
            write_wait(f, f % NBUF)

    return gather_kernel, NW, nchunks, C


def kernel(x_word, table):
    B, S = x_word.shape
    V, D = table.shape
    N = B * S
    info = plsc.get_sparse_core_info()
    fn, NW, nchunks, C = _make_gather(N, V, D, info.num_cores,
                                      info.num_subcores)
    idx = x_word.reshape(NW, nchunks, C).astype(jnp.int32)
    pch = (nchunks + 7) // 8 * 8
    idx = jnp.pad(idx, ((0, 0), (0, pch - nchunks), (0, 0)))
    out = fn(idx, table)
    return out.reshape(B, S, D)


# speedup vs baseline: 10.3872x; 1.0013x over previous
import functools

import jax
import jax.numpy as jnp
from jax import lax
from jax.experimental import pallas as pl
from jax.experimental.pallas import tpu as pltpu
from jax.experimental.pallas import tpu_sc as plsc


def _make_gather(N, V, D, NC, NS):
    NW = NC * NS
    C = 128
    K = 2
    NBUF = 3
    DW = 1
    n_per_w = N // NW
    nchunks = n_per_w // C
    nfills = nchunks // K
    assert n_per_w % (K * C) == 0 and nfills > NBUF
    nfills_main = nfills - (nfills % NBUF)
    PCH = (nchunks + 7) // 8 * 8
    mesh = plsc.VectorSubcoreMesh(core_axis_name="c", subcore_axis_name="s")

    @functools.partial(
        pl.kernel,
        mesh=mesh,
        out_type=jax.ShapeDtypeStruct((N, D), jnp.float32),
        scratch_types=[
            pltpu.VMEM((PCH, C), jnp.int32),
            [pltpu.VMEM((K * C, D), jnp.float32)] * NBUF,
            [pltpu.SemaphoreType.DMA] * NBUF,
            [pltpu.SemaphoreType.DMA] * NBUF,
        ],
    )
    def gather_kernel(idx_hbm, table_hbm, out_hbm, idx_v, rows, gsems, wsems):
        wid = lax.axis_index("s") * NC + lax.axis_index("c")
        row_base = wid * n_per_w
        pltpu.sync_copy(idx_hbm.at[wid], idx_v)

        def gather_start(f, b):
            for t in range(K):
                pltpu.async_copy(table_hbm.at[idx_v.at[K * f + t]],
                                 rows[b].at[pl.ds(t * C, C)], gsems[b])

        def gather_wait(f, b):
            for t in range(K):
                pltpu.make_async_copy(table_hbm.at[idx_v.at[K * f + t]],
                                      rows[b].at[pl.ds(t * C, C)],
                                      gsems[b]).wait()

        def write_start(f, b):
            pltpu.async_copy(
                rows[b], out_hbm.at[pl.ds(row_base + f * K * C, K * C)],
                wsems[b])

        def write_wait(f, b):
            pltpu.make_async_copy(
                rows[b], out_hbm.at[pl.ds(row_base + f * K * C, K * C)],
                wsems[b]).wait()

        for j in range(NBUF - DW):
            gather_start(j, j)

        @pl.loop(0, nfills_main, step=NBUF)
        def _(g):
            for j in range(NBUF):
                f = g + j
                gather_wait(f, j)
                write_start(f, j)
                b2 = (j - DW) % NBUF

                @pl.when(f >= DW)
                def _():
                    write_wait(f - DW, b2)

                @pl.when(f + (NBUF - DW) < nfills)
                def _():
                    gather_start(f + (NBUF - DW), b2)

        for f in range(nfills_main, nfills):
            gather_wait(f, f % NBUF)
            write_start(f, f % NBUF)
            write_wait(f - DW, (f - DW) % NBUF)
        for t in range(DW):
            f = nfills - DW + t
            write_wait(f, f % NBUF)

    return gather_kernel, NW, nchunks, C


def kernel(x_word, table):
    B, S = x_word.shape
    V, D = table.shape
    N = B * S
    info = plsc.get_sparse_core_info()
    fn, NW, nchunks, C = _make_gather(N, V, D, info.num_cores,
                                      info.num_subcores)
    idx = x_word.T.reshape(NW, nchunks, C).astype(jnp.int32)
    pch = (nchunks + 7) // 8 * 8
    idx = jnp.pad(idx, ((0, 0), (0, pch - nchunks), (0, 0)))
    out = fn(idx, table)
    return out.reshape(S, B, D).transpose(1, 0, 2)
